# baseline (device time: 36635 ns/iter reference)
import jax
import jax.numpy as jnp
from jax import lax
from jax.experimental import pallas as pl
from jax.experimental.pallas import tpu as pltpu

N_DEV = 16
SQ = 256
D = 1024
DH = 128
NH_LOCAL = 8
CH = SQ // N_DEV
SCALE = 0.08838834764831843


def kernel(x, Wq, Wo, Wk, Wv):
    def body(
        x_ref,
        wq_ref,
        wo_ref,
        wk_ref,
        wv_ref,
        out_ref,
        out_vm,
        part16_ref,
        rs_buf,
        ag_buf,
        store_sem,
        rs_send_sems,
        rs_recv_sems,
        ag_send_sems,
        ag_recv_sems,
    ):
        p = lax.axis_index("i")
        bf16 = jnp.bfloat16

        barrier = pltpu.get_barrier_semaphore()
        for j in range(N_DEV - 1):
            pl.semaphore_signal(
                barrier,
                inc=1,
                device_id=(lax.rem(p + 1 + j, N_DEV),),
                device_id_type=pl.DeviceIdType.MESH,
            )

        xm = x_ref[0].astype(bf16)
        q = jnp.dot(xm, wq_ref[...].astype(bf16), preferred_element_type=jnp.float32)
        k = jnp.dot(xm, wk_ref[...].astype(bf16), preferred_element_type=jnp.float32)
        v = jnp.dot(xm, wv_ref[...].astype(bf16), preferred_element_type=jnp.float32)
        q16 = q.astype(bf16)
        k16 = k.astype(bf16)
        v16 = v.astype(bf16)
        wo16 = wo_ref[...].astype(bf16)

        part = jnp.zeros((SQ, D), jnp.float32)
        for h in range(NH_LOCAL):
            sl = slice(h * DH, (h + 1) * DH)
            s = (
                lax.dot_general(
                    q16[:, sl],
                    k16[:, sl],
                    (((1,), (1,)), ((), ())),
                    preferred_element_type=jnp.float32,
                )
                * SCALE
            )
            m = jnp.max(s, axis=1, keepdims=True)
            pr = jnp.exp(s - m)
            l = jnp.sum(pr, axis=1, keepdims=True)
            o = jnp.dot(
                pr.astype(bf16), v16[:, sl], preferred_element_type=jnp.float32
            ) / l
            part = part + jnp.dot(
                o.astype(bf16), wo16[sl, :], preferred_element_type=jnp.float32
            )
        part16_ref[...] = part.astype(bf16)
        rs_buf[p, :, :] = part16_ref[pl.ds(p * CH, CH), :]

        pl.semaphore_wait(barrier, N_DEV - 1)

        NS = 4
        HC = D // NS
        myrows = pl.ds(p * CH, CH)
        rs_rdmas = {h: [] for h in range(NS)}
        for half in range(NS):
            cols = pl.ds(half * HC, HC)
            for j in range(N_DEV - 1):
                tgt = lax.rem(p + 1 + j, N_DEV)
                rdma = pltpu.make_async_remote_copy(
                    src_ref=part16_ref.at[pl.ds(tgt * CH, CH), cols],
                    dst_ref=rs_buf.at[p, :, cols],
                    send_sem=rs_send_sems.at[half, j],
                    recv_sem=rs_recv_sems.at[half, j],
                    device_id=(tgt,),
                    device_id_type=pl.DeviceIdType.MESH,
                )
                rdma.start()
                rs_rdmas[half].append(rdma)

        ag_rdmas = []
        reds = []
        for half in range(NS):
            cols = pl.ds(half * HC, HC)
            for rdma in rs_rdmas[half]:
                rdma.wait_recv()
            red = rs_buf[0, :, cols].astype(jnp.float32)
            for s_ in range(1, N_DEV):
                red = red + rs_buf[s_, :, cols].astype(jnp.float32)
            reds.append(red)
            ag_buf[myrows, cols] = red.astype(bf16)
            for j in range(N_DEV - 1):
                tgt = lax.rem(p + 1 + j, N_DEV)
                rdma = pltpu.make_async_remote_copy(
                    src_ref=ag_buf.at[myrows, cols],
                    dst_ref=ag_buf.at[myrows, cols],
                    send_sem=ag_send_sems.at[half, j],
                    recv_sem=ag_recv_sems.at[half, j],
                    device_id=(tgt,),
                    device_id_type=pl.DeviceIdType.MESH,
                )
                rdma.start()
                ag_rdmas.append(rdma)
        for rdma in ag_rdmas:
            rdma.wait_recv()

        out_vm[0] = ag_buf[...].astype(jnp.float32)
        for half in range(NS):
            out_vm[0, myrows, half * HC : (half + 1) * HC] = reds[half]
        store = pltpu.make_async_copy(out_vm, out_ref, store_sem)
        store.start()
        store.wait()

        for half in range(NS):
            for rdma in rs_rdmas[half]:
                rdma.wait_send()
        for rdma in ag_rdmas:
            rdma.wait_send()

    return pl.pallas_call(
        body,
        out_shape=jax.ShapeDtypeStruct((1, SQ, D), jnp.float32),
        in_specs=[pl.BlockSpec(memory_space=pltpu.VMEM)] * 5,
        out_specs=pl.BlockSpec(memory_space=pltpu.MemorySpace.HBM),
        scratch_shapes=[
            pltpu.VMEM((1, SQ, D), jnp.float32),
            pltpu.VMEM((SQ, D), jnp.bfloat16),
            pltpu.VMEM((N_DEV, CH, D), jnp.bfloat16),
            pltpu.VMEM((SQ, D), jnp.bfloat16),
            pltpu.SemaphoreType.DMA,
            pltpu.SemaphoreType.DMA((4, N_DEV - 1)),
            pltpu.SemaphoreType.DMA((4, N_DEV - 1)),
            pltpu.SemaphoreType.DMA((4, N_DEV - 1)),
            pltpu.SemaphoreType.DMA((4, N_DEV - 1)),
        ],
        compiler_params=pltpu.CompilerParams(collective_id=0),
    )(x, Wq, Wo, Wk, Wv)


# device time: 34606 ns/iter; 1.0586x vs baseline; 1.0586x over previous
import jax
import jax.numpy as jnp
from jax import lax
from jax.experimental import pallas as pl
from jax.experimental.pallas import tpu as pltpu

N_DEV = 16
SQ = 256
D = 1024
DH = 128
NH_LOCAL = 8
CH = SQ // N_DEV
SCALE = 0.08838834764831843


def kernel(x, Wq, Wo, Wk, Wv):
    def body(
        x_ref,
        wq_ref,
        wo_ref,
        wk_ref,
        wv_ref,
        out_ref,
        out_vm,
        part16_ref,
        rs_buf,
        ag_buf,
        store_sem,
        rs_send_sems,
        rs_recv_sems,
        ag_send_sems,
        ag_recv_sems,
    ):
        p = lax.axis_index("i")
        bf16 = jnp.bfloat16

        barrier = pltpu.get_barrier_semaphore()
        for j in range(N_DEV - 1):
            pl.semaphore_signal(
                barrier,
                inc=1,
                device_id=(lax.rem(p + 1 + j, N_DEV),),
                device_id_type=pl.DeviceIdType.MESH,
            )

        xm = x_ref[0].astype(bf16)
        q = jnp.dot(xm, wq_ref[...].astype(bf16), preferred_element_type=jnp.float32)
        k = jnp.dot(xm, wk_ref[...].astype(bf16), preferred_element_type=jnp.float32)
        v = jnp.dot(xm, wv_ref[...].astype(bf16), preferred_element_type=jnp.float32)
        q16 = q.astype(bf16)
        k16 = k.astype(bf16)
        v16 = v.astype(bf16)
        wo16 = wo_ref[...].astype(bf16)

        part = jnp.zeros((SQ, D), jnp.float32)
        for h in range(NH_LOCAL):
            sl = slice(h * DH, (h + 1) * DH)
            s = (
                lax.dot_general(
                    q16[:, sl],
                    k16[:, sl],
                    (((1,), (1,)), ((), ())),
                    preferred_element_type=jnp.float32,
                )
                * SCALE
            )
            m = jnp.max(s, axis=1, keepdims=True)
            pr = jnp.exp(s - m)
            l = jnp.sum(pr, axis=1, keepdims=True)
            o = jnp.dot(
                pr.astype(bf16), v16[:, sl], preferred_element_type=jnp.float32
            ) / l
            part = part + jnp.dot(
                o.astype(bf16), wo16[sl, :], preferred_element_type=jnp.float32
            )
        part16_ref[...] = part.astype(bf16)
        rs_buf[p, :, :] = part16_ref[pl.ds(p * CH, CH), :]

        pl.semaphore_wait(barrier, N_DEV - 1)

        HC = D // 2
        myrows = pl.ds(p * CH, CH)
        rs_rdmas = {0: [], 1: []}
        for half in (0, 1):
            cols = pl.ds(half * HC, HC)
            for j in range(N_DEV - 1):
                tgt = lax.rem(p + 1 + j, N_DEV)
                rdma = pltpu.make_async_remote_copy(
                    src_ref=part16_ref.at[pl.ds(tgt * CH, CH), cols],
                    dst_ref=rs_buf.at[p, :, cols],
                    send_sem=rs_send_sems.at[half, j],
                    recv_sem=rs_recv_sems.at[half, j],
                    device_id=(tgt,),
                    device_id_type=pl.DeviceIdType.MESH,
                )
                rdma.start()
                rs_rdmas[half].append(rdma)

        ag_rdmas = {0: [], 1: []}
        reds = []
        for half in (0, 1):
            cols = pl.ds(half * HC, HC)
            for rdma in rs_rdmas[half]:
                rdma.wait_recv()
            vals = [rs_buf[s_, :, cols].astype(jnp.float32) for s_ in range(N_DEV)]
            while len(vals) > 1:
                vals = [
                    vals[i] + vals[i + 1] for i in range(0, len(vals), 2)
                ]
            red = vals[0]
            reds.append(red)
            ag_buf[myrows, cols] = red.astype(bf16)
            for j in range(N_DEV - 1):
                tgt = lax.rem(p + 1 + j, N_DEV)
                rdma = pltpu.make_async_remote_copy(
                    src_ref=ag_buf.at[myrows, cols],
                    dst_ref=ag_buf.at[myrows, cols],
                    send_sem=ag_send_sems.at[half, j],
                    recv_sem=ag_recv_sems.at[half, j],
                    device_id=(tgt,),
                    device_id_type=pl.DeviceIdType.MESH,
                )
                rdma.start()
                ag_rdmas[half].append(rdma)

        stores = []
        for half in (0, 1):
            cols = pl.ds(half * HC, HC)
            for rdma in ag_rdmas[half]:
                rdma.wait_recv()
            out_vm[0, :, cols] = ag_buf[:, cols].astype(jnp.float32)
            out_vm[0, myrows, cols] = reds[half]
            store = pltpu.make_async_copy(
                out_vm.at[:, :, cols],
                out_ref.at[:, :, cols],
                store_sem.at[half],
            )
            store.start()
            stores.append(store)
        for store in stores:
            store.wait()

        for rdma in rs_rdmas[0] + rs_rdmas[1]:
            rdma.wait_send()
        for rdma in ag_rdmas[0] + ag_rdmas[1]:
            rdma.wait_send()

    return pl.pallas_call(
        body,
        out_shape=jax.ShapeDtypeStruct((1, SQ, D), jnp.float32),
        in_specs=[pl.BlockSpec(memory_space=pltpu.VMEM)] * 5,
        out_specs=pl.BlockSpec(memory_space=pltpu.MemorySpace.HBM),
        scratch_shapes=[
            pltpu.VMEM((1, SQ, D), jnp.float32),
            pltpu.VMEM((SQ, D), jnp.bfloat16),
            pltpu.VMEM((N_DEV, CH, D), jnp.bfloat16),
            pltpu.VMEM((SQ, D), jnp.bfloat16),
            pltpu.SemaphoreType.DMA((2,)),
            pltpu.SemaphoreType.DMA((2, N_DEV - 1)),
            pltpu.SemaphoreType.DMA((2, N_DEV - 1)),
            pltpu.SemaphoreType.DMA((2, N_DEV - 1)),
            pltpu.SemaphoreType.DMA((2, N_DEV - 1)),
        ],
        compiler_params=pltpu.CompilerParams(collective_id=0),
    )(x, Wq, Wo, Wk, Wv)
